# baseline (speedup 1.0000x reference)
import functools

import jax
import jax.numpy as jnp
from jax.experimental import pallas as pl
from jax.experimental.pallas import tpu as pltpu

N = 4096
D = 256
BLK = 512
DE = D + 128
CHUNK = 1024


def _gat_layer_kernel(x_ref, w_ref, asrc_ref, adst_ref, adj_ref, out_ref,
                      h_ref, hb_ref, dt_ref):
    i = pl.program_id(0)

    @pl.when(i == 0)
    def _():
        h = jnp.dot(x_ref[...], w_ref[...], preferred_element_type=jnp.float32)
        h_ref[...] = h
        hb_ref[:, :D] = h.astype(jnp.bfloat16)
        hb_ref[:, D:] = jnp.ones((N, DE - D), jnp.bfloat16)
        d_t = jax.lax.dot_general(
            adst_ref[...], h,
            dimension_numbers=(((0,), (1,)), ((), ())),
            preferred_element_type=jnp.float32)
        d_c = jnp.minimum(d_t, 40.0)
        dt_ref[...] = jnp.concatenate(
            [jnp.broadcast_to(d_c, (4, N)),
             jnp.broadcast_to(0.2 * d_c, (4, N))], axis=0)

    h_blk = h_ref[pl.ds(i * BLK, BLK), :]
    s = jnp.minimum(
        jnp.dot(h_blk, asrc_ref[...], preferred_element_type=jnp.float32),
        40.0)
    s2 = 0.2 * s
    acc = jnp.zeros((BLK, DE), jnp.float32)
    for c in range(N // CHUNK):
        lo = c * CHUNK
        e = jnp.maximum(s + dt_ref[0:1, pl.ds(lo, CHUNK)],
                        s2 + dt_ref[4:5, pl.ds(lo, CHUNK)])
        p = adj_ref[:, pl.ds(lo, CHUNK)] * jnp.exp(e)
        acc = acc + jnp.dot(p.astype(jnp.bfloat16), hb_ref[pl.ds(lo, CHUNK), :],
                            preferred_element_type=jnp.float32)
    agg = acc[:, :D] / acc[:, D:D + 1]
    out_ref[...] = jnp.where(agg > 0, agg, jnp.exp(agg) - 1.0)


def _gat_layer(x, adj, W, a_src, a_dst):
    grid = (N // BLK,)
    return pl.pallas_call(
        _gat_layer_kernel,
        grid=grid,
        in_specs=[
            pl.BlockSpec((N, D), lambda i: (0, 0)),
            pl.BlockSpec((D, D), lambda i: (0, 0)),
            pl.BlockSpec((D, 1), lambda i: (0, 0)),
            pl.BlockSpec((D, 1), lambda i: (0, 0)),
            pl.BlockSpec((BLK, N), lambda i: (i, 0)),
        ],
        out_specs=pl.BlockSpec((BLK, D), lambda i: (i, 0)),
        out_shape=jax.ShapeDtypeStruct((N, D), jnp.float32),
        scratch_shapes=[pltpu.VMEM((N, D), jnp.float32),
                        pltpu.VMEM((N, DE), jnp.bfloat16),
                        pltpu.VMEM((8, N), jnp.float32)],
    )(x, W, a_src, a_dst, adj)


@jax.jit
def kernel(inputs, adjacency_matrix, W1, a_src1, a_dst1, W2, a_src2, a_dst2):
    x = _gat_layer(inputs, adjacency_matrix, W1, a_src1, a_dst1)
    x = _gat_layer(x, adjacency_matrix, W2, a_src2, a_dst2)
    return x


# speedup vs baseline: 1.0012x; 1.0012x over previous
import functools

import jax
import jax.numpy as jnp
from jax.experimental import pallas as pl
from jax.experimental.pallas import tpu as pltpu

N = 4096
D = 256
BLK = 512
DE = D + 128
CHUNK = 512


def _gat_layer_kernel(x_ref, w_ref, asrc_ref, adst_ref, adj_ref, out_ref,
                      h_ref, hb_ref, dt_ref):
    i = pl.program_id(0)

    @pl.when(i == 0)
    def _():
        h = jnp.dot(x_ref[...], w_ref[...], preferred_element_type=jnp.float32)
        h_ref[...] = h
        hb_ref[:, :D] = h.astype(jnp.bfloat16)
        hb_ref[:, D:] = jnp.ones((N, DE - D), jnp.bfloat16)
        d_t = jax.lax.dot_general(
            adst_ref[...], h,
            dimension_numbers=(((0,), (1,)), ((), ())),
            preferred_element_type=jnp.float32)
        d_c = jnp.minimum(d_t, 40.0)
        dt_ref[...] = jnp.concatenate(
            [jnp.broadcast_to(d_c, (4, N)),
             jnp.broadcast_to(0.2 * d_c, (4, N))], axis=0)

    h_blk = h_ref[pl.ds(i * BLK, BLK), :]
    s = jnp.minimum(
        jnp.dot(h_blk, asrc_ref[...], preferred_element_type=jnp.float32),
        40.0)
    s2 = 0.2 * s
    acc = jnp.zeros((BLK, DE), jnp.float32)
    for c in range(N // CHUNK):
        lo = c * CHUNK
        e = jnp.maximum(s + dt_ref[0:1, pl.ds(lo, CHUNK)],
                        s2 + dt_ref[4:5, pl.ds(lo, CHUNK)])
        p = adj_ref[:, pl.ds(lo, CHUNK)] * jnp.exp(e)
        acc = acc + jnp.dot(p.astype(jnp.bfloat16), hb_ref[pl.ds(lo, CHUNK), :],
                            preferred_element_type=jnp.float32)
    agg = acc[:, :D] / acc[:, D:D + 1]
    out_ref[...] = jnp.where(agg > 0, agg, jnp.exp(agg) - 1.0)


def _gat_layer(x, adj, W, a_src, a_dst):
    grid = (N // BLK,)
    return pl.pallas_call(
        _gat_layer_kernel,
        grid=grid,
        in_specs=[
            pl.BlockSpec((N, D), lambda i: (0, 0)),
            pl.BlockSpec((D, D), lambda i: (0, 0)),
            pl.BlockSpec((D, 1), lambda i: (0, 0)),
            pl.BlockSpec((D, 1), lambda i: (0, 0)),
            pl.BlockSpec((BLK, N), lambda i: (i, 0)),
        ],
        out_specs=pl.BlockSpec((BLK, D), lambda i: (i, 0)),
        out_shape=jax.ShapeDtypeStruct((N, D), jnp.float32),
        scratch_shapes=[pltpu.VMEM((N, D), jnp.float32),
                        pltpu.VMEM((N, DE), jnp.bfloat16),
                        pltpu.VMEM((8, N), jnp.float32)],
    )(x, W, a_src, a_dst, adj)


@jax.jit
def kernel(inputs, adjacency_matrix, W1, a_src1, a_dst1, W2, a_src2, a_dst2):
    x = _gat_layer(inputs, adjacency_matrix, W1, a_src1, a_dst1)
    x = _gat_layer(x, adjacency_matrix, W2, a_src2, a_dst2)
    return x
